# Initial kernel scaffold; baseline (speedup 1.0000x reference)
import jax
import jax.numpy as jnp
from jax.experimental import pallas as pl


def kernel(embedding, reservoir_encoded, base_indices):
    raise NotImplementedError("write your pallas kernel here")


# speedup vs baseline: 12.4878x; 12.4878x over previous
import jax
import jax.numpy as jnp
from jax import lax
from jax.experimental import pallas as pl
from jax.experimental.pallas import tpu as pltpu
from jax.experimental.pallas import tpu_sc as plsc

VOCAB, F = 30522, 64
NT, R = 100000, 8
B, L = 4096, 200
T = B * L
NC, NS, LANES = 2, 16, 16
NW = NC * NS
TPW = T // NW
C = 128
G = TPW // C
CR = C * R


def _body(emb, res, bidx, out, bidx_v, ids_v, fidx_v, rows_v, slot_v, zero_v,
          acc_sh):
    cid = lax.axis_index("c")
    sid = lax.axis_index("s")
    wid = sid * NC + cid

    zv = jnp.zeros((LANES,), jnp.float32)

    def zero_body(t, _):
        for j in range(F // LANES):
            zero_v[t, pl.ds(j * LANES, LANES)] = zv
        return ()

    lax.fori_loop(0, C, zero_body, ())

    iota = lax.iota(jnp.int32, LANES)
    hi = lax.shift_right_logical(iota, 3)
    col = lax.bitwise_and(iota, 7)
    acc_base = sid * C

    def chunk_body(g, _):
        tok0 = wid * TPW + g * C
        pltpu.sync_copy(bidx.at[pl.ds(tok0, C)], bidx_v)
        pltpu.sync_copy(res.at[bidx_v], ids_v)
        for k in range(CR // LANES):
            ids16 = plsc.load_gather(ids_v, [hi + 2 * k, col])
            fidx_v[pl.ds(k * LANES, LANES)] = ids16
            base = hi + (acc_base + 2 * k)
            slot_v[pl.ds(k * LANES, LANES)] = jnp.where(ids16 == 0, -1, base)
        pltpu.sync_copy(emb.at[fidx_v], rows_v)
        pltpu.sync_copy(zero_v, acc_sh.at[pl.ds(acc_base, C), :])
        pltpu.sync_copy(
            rows_v, acc_sh.at[plsc.Indices(slot_v, ignored_value=-1)], add=True
        )
        pltpu.sync_copy(acc_sh.at[pl.ds(acc_base, C), :], out.at[pl.ds(tok0, C), :])
        return ()

    lax.fori_loop(0, G, chunk_body, ())


@jax.jit
def kernel(embedding, reservoir_encoded, base_indices):
    mesh = plsc.VectorSubcoreMesh(core_axis_name="c", subcore_axis_name="s")
    run = pl.kernel(
        _body,
        out_type=jax.ShapeDtypeStruct((T, F), jnp.float32),
        mesh=mesh,
        compiler_params=pltpu.CompilerParams(
            needs_layout_passes=False, use_tc_tiling_on_sc=False
        ),
        scratch_types=[
            pltpu.VMEM((C,), jnp.int32),
            pltpu.VMEM((C, R), jnp.int32),
            pltpu.VMEM((CR,), jnp.int32),
            pltpu.VMEM((CR, F), jnp.float32),
            pltpu.VMEM((CR,), jnp.int32),
            pltpu.VMEM((C, F), jnp.float32),
            pltpu.VMEM_SHARED((NS * C, F), jnp.float32),
        ],
    )
    out = run(embedding, reservoir_encoded, base_indices.reshape(T))
    return out.reshape(B, L, F)


# speedup vs baseline: 19.2488x; 1.5414x over previous
import jax
import jax.numpy as jnp
from jax import lax
from jax.experimental import pallas as pl
from jax.experimental.pallas import tpu as pltpu
from jax.experimental.pallas import tpu_sc as plsc

VOCAB, F = 30522, 64
NT, R = 100000, 8
B, L = 4096, 200
T = B * L
NC, NS, LANES = 2, 16, 16
NW = NC * NS
TPW = T // NW
C = 128
G = TPW // C
CR = C * R
KPJ = C // LANES


def _body(emb, res, bidx, out, bidx2, ids2, fidx2, acc2, sem_ids, sem_add,
          sem_out):
    cid = lax.axis_index("c")
    sid = lax.axis_index("s")
    wid = sid * NC + cid
    base = wid * TPW

    iota = lax.iota(jnp.int32, LANES)
    zv = jnp.zeros((LANES,), jnp.float32)

    def front(g, p):
        tok0 = base + g * C
        pltpu.sync_copy(bidx.at[pl.ds(tok0, C)], bidx2.at[p])
        pltpu.async_copy(res.at[bidx2.at[p]], ids2.at[p], sem_ids)

    def wait_ids(p):
        pltpu.make_async_copy(res.at[bidx2.at[p]], ids2.at[p], sem_ids).wait()

    def build(p):
        for k in range(R * KPJ):
            j = k // KPJ
            rows = iota + (k % KPJ) * LANES
            cols = jnp.full((LANES,), j, jnp.int32)
            ids16 = plsc.load_gather(ids2.at[p], [rows, cols])
            fidx2[p, pl.ds(k * LANES, LANES)] = jnp.where(ids16 == 0, -1, ids16)

    def zero_acc(p):
        def zb(t, _):
            for jj in range(F // LANES):
                acc2[p, t, pl.ds(jj * LANES, LANES)] = zv
            return ()

        lax.fori_loop(0, C, zb, ())

    def gathers(p):
        descs = []
        for j in range(R):
            idx = plsc.Indices(
                fidx2.at[p].at[pl.ds(j * C, C)], ignored_value=-1
            )
            descs.append(
                pltpu.async_copy(emb.at[idx], acc2.at[p], sem_add, add=True)
            )
        for d in descs:
            d.wait()

    def out_issue(g, p):
        pltpu.async_copy(acc2.at[p], out.at[pl.ds(base + g * C, C), :], sem_out)

    def wait_out(g, p):
        pltpu.make_async_copy(
            acc2.at[p], out.at[pl.ds(base + g * C, C), :], sem_out
        ).wait()

    front(0, 0)

    def loop_body(i, _):
        for ph in range(2):
            g = 2 * i + ph
            p = ph

            wait_ids(p)
            build(p)

            @pl.when(g + 1 < G)
            def _next_front():
                front(g + 1, 1 - p)

            @pl.when(g >= 2)
            def _reclaim_acc():
                wait_out(g - 2, p)

            zero_acc(p)
            gathers(p)
            out_issue(g, p)
        return ()

    lax.fori_loop(0, G // 2, loop_body, ())
    wait_out(G - 2, 0)
    wait_out(G - 1, 1)


@jax.jit
def kernel(embedding, reservoir_encoded, base_indices):
    mesh = plsc.VectorSubcoreMesh(core_axis_name="c", subcore_axis_name="s")
    run = pl.kernel(
        _body,
        out_type=jax.ShapeDtypeStruct((T, F), jnp.float32),
        mesh=mesh,
        compiler_params=pltpu.CompilerParams(
            needs_layout_passes=False, use_tc_tiling_on_sc=False
        ),
        scratch_types=[
            pltpu.VMEM((2, C), jnp.int32),
            pltpu.VMEM((2, C, R), jnp.int32),
            pltpu.VMEM((2, CR), jnp.int32),
            pltpu.VMEM((2, C, F), jnp.float32),
            pltpu.SemaphoreType.DMA,
            pltpu.SemaphoreType.DMA,
            pltpu.SemaphoreType.DMA,
        ],
    )
    out = run(embedding, reservoir_encoded, base_indices.reshape(T))
    return out.reshape(B, L, F)


# speedup vs baseline: 23.0190x; 1.1959x over previous
import jax
import jax.numpy as jnp
from jax import lax
from jax.experimental import pallas as pl
from jax.experimental.pallas import tpu as pltpu
from jax.experimental.pallas import tpu_sc as plsc

VOCAB, F = 30522, 64
NT, R = 100000, 8
B, L = 4096, 200
T = B * L
NC, NS, LANES = 2, 16, 16
NW = NC * NS
TPW = T // NW
C = 256
G = TPW // C
CR = C * R
KPJ = C // LANES


def _body(emb, res, bidx, out, bidx2, ids2, fidx2, acc2, sem_ids, sem_add,
          sem_out):
    cid = lax.axis_index("c")
    sid = lax.axis_index("s")
    wid = sid * NC + cid
    base = wid * TPW

    iota = lax.iota(jnp.int32, LANES)
    zv = jnp.zeros((LANES,), jnp.float32)

    def front(g, p):
        tok0 = base + g * C
        pltpu.sync_copy(bidx.at[pl.ds(tok0, C)], bidx2.at[p])
        pltpu.async_copy(res.at[bidx2.at[p]], ids2.at[p], sem_ids)

    def wait_ids(p):
        pltpu.make_async_copy(res.at[bidx2.at[p]], ids2.at[p], sem_ids).wait()

    def build(p):
        for k in range(R * KPJ):
            j = k // KPJ
            rows = iota + (k % KPJ) * LANES
            cols = jnp.full((LANES,), j, jnp.int32)
            ids16 = plsc.load_gather(ids2.at[p], [rows, cols])
            fidx2[p, pl.ds(k * LANES, LANES)] = jnp.where(ids16 == 0, -1, ids16)

    def zero_acc(p):
        def zb(t, _):
            for jj in range(F // LANES):
                acc2[p, t, pl.ds(jj * LANES, LANES)] = zv
            return ()

        lax.fori_loop(0, C, zb, ())

    def fire_adds(p):
        descs = []
        for j in range(R):
            idx = plsc.Indices(
                fidx2.at[p].at[pl.ds(j * C, C)], ignored_value=-1
            )
            descs.append(
                pltpu.async_copy(emb.at[idx], acc2.at[p], sem_add, add=True)
            )
        return descs

    def out_issue(g, p):
        pltpu.async_copy(acc2.at[p], out.at[pl.ds(base + g * C, C), :], sem_out)

    def wait_out(g, p):
        pltpu.make_async_copy(
            acc2.at[p], out.at[pl.ds(base + g * C, C), :], sem_out
        ).wait()

    front(0, 0)
    wait_ids(0)
    build(0)
    zero_acc(0)

    def loop_body(i, _):
        for ph in range(2):
            g = 2 * i + ph
            p = ph

            @pl.when(g + 1 < G)
            def _next_front():
                front(g + 1, 1 - p)

            descs = fire_adds(p)

            @pl.when(g + 1 < G)
            def _next_prep():
                wait_ids(1 - p)
                build(1 - p)

                @pl.when(g >= 1)
                def _reclaim_acc():
                    wait_out(g - 1, 1 - p)

                zero_acc(1 - p)

            for d in descs:
                d.wait()
            out_issue(g, p)
        return ()

    lax.fori_loop(0, G // 2, loop_body, ())
    wait_out(G - 2, 0)
    wait_out(G - 1, 1)


@jax.jit
def kernel(embedding, reservoir_encoded, base_indices):
    mesh = plsc.VectorSubcoreMesh(core_axis_name="c", subcore_axis_name="s")
    run = pl.kernel(
        _body,
        out_type=jax.ShapeDtypeStruct((T, F), jnp.float32),
        mesh=mesh,
        compiler_params=pltpu.CompilerParams(
            needs_layout_passes=False, use_tc_tiling_on_sc=False
        ),
        scratch_types=[
            pltpu.VMEM((2, C), jnp.int32),
            pltpu.VMEM((2, C, R), jnp.int32),
            pltpu.VMEM((2, CR), jnp.int32),
            pltpu.VMEM((2, C, F), jnp.float32),
            pltpu.SemaphoreType.DMA,
            pltpu.SemaphoreType.DMA,
            pltpu.SemaphoreType.DMA,
        ],
    )
    out = run(embedding, reservoir_encoded, base_indices.reshape(T))
    return out.reshape(B, L, F)
